# baseline (speedup 1.0000x reference)
import functools

import jax
import jax.numpy as jnp
from jax import lax
from jax.experimental import pallas as pl
from jax.experimental.pallas import tpu as pltpu
from jax.experimental.pallas import tpu_sc as plsc

N, K, M = 10000, 64, 200


_BN = 80


def _adj_body(sa_ref, sba_ref, adj_ref):
    x = sba_ref[...]
    mx = jnp.max(x, axis=2, keepdims=True)
    s = jnp.sum(jnp.exp(x - mx), axis=2)
    lse_b = mx[:, :, 0] + jnp.log(s)
    y = sa_ref[...]
    my = jnp.max(y, axis=1, keepdims=True)
    sy = jnp.sum(jnp.exp(y - my), axis=1, keepdims=True)
    lse_a = my + jnp.log(sy)
    adj_ref[...] = y - lse_a - lse_b


def _compute_adj(sa, sba):
    return pl.pallas_call(
        _adj_body,
        grid=(N // _BN,),
        in_specs=[
            pl.BlockSpec((_BN, K), lambda i: (i, 0)),
            pl.BlockSpec((_BN, K, K), lambda i: (i, 0, 0)),
        ],
        out_specs=pl.BlockSpec((_BN, K), lambda i: (i, 0)),
        out_shape=jax.ShapeDtypeStruct((N, K), jnp.float32),
    )(sa, sba)


_NC, _NS, _L = 2, 16, 16
_NW = _NC * _NS
_R = 8
_C = N // _R
_T = (_C + _NW - 1) // _NW
_OFFS = tuple(j * _L for j in range(M // _L)) + (M - _L,)


def _gather_call(sba, a, b, adj):
    mesh = plsc.VectorSubcoreMesh(core_axis_name="c", subcore_axis_name="s")

    @functools.partial(
        pl.kernel,
        mesh=mesh,
        compiler_params=pltpu.CompilerParams(needs_layout_passes=False),
        out_type=jax.ShapeDtypeStruct((N * M,), jnp.float32),
        scratch_types=[
            pltpu.VMEM((_R, K, K), jnp.float32),
            pltpu.VMEM((_R, M), jnp.int32),
            pltpu.VMEM((_R, M), jnp.int32),
            pltpu.VMEM((_R, K), jnp.float32),
            pltpu.VMEM((_R * M,), jnp.float32),
        ],
    )
    def k(sba_h, a_h, b_h, adj_h, out_h, sba_v, a_v, b_v, adj_v, out_v):
        wid = lax.axis_index("s") * _NC + lax.axis_index("c")

        def trip(i, carry):
            ci = i * _NW + wid

            @pl.when(ci < _C)
            def _():
                r0 = ci * _R
                pltpu.sync_copy(sba_h.at[pl.ds(r0, _R)], sba_v)
                pltpu.sync_copy(a_h.at[pl.ds(r0, _R)], a_v)
                pltpu.sync_copy(b_h.at[pl.ds(r0, _R)], b_v)
                pltpu.sync_copy(adj_h.at[pl.ds(r0, _R)], adj_v)
                for r in range(_R):
                    rv = jnp.full((_L,), r, jnp.int32)
                    for off in _OFFS:
                        av = a_v[r, pl.ds(off, _L)]
                        bv = b_v[r, pl.ds(off, _L)]
                        g = plsc.load_gather(sba_v, [rv, av, bv])
                        adjv = plsc.load_gather(adj_v, [rv, av])
                        out_v[pl.ds(r * M + off, _L)] = g + adjv
                pltpu.sync_copy(out_v, out_h.at[pl.ds(ci * (_R * M), _R * M)])

            return carry

        lax.fori_loop(0, _T, trip, 0)

    return k(sba, a, b, adj)


def kernel(a, b, sa, sba):
    adj = _compute_adj(sa, sba)
    return _gather_call(sba, a.astype(jnp.int32), b.astype(jnp.int32), adj)
